# initial kernel scaffold (unmeasured)
import jax
import jax.numpy as jnp
from jax import lax
from jax.experimental import pallas as pl
from jax.experimental.pallas import tpu as pltpu

N_DEV = 4
HQ = 8
H_ALL = 32
DH = 128
SQ = 2048
NCLS = 4
NG = 8
BLK = 64
CLS_ROWS = NG * BLK
QUARTER = SQ // N_DEV
DM = 1024
SCALE = 0.08838834764831843


def _body(x_ref, wq_ref, kp_ref, vp_ref, wo_ref, out_ref,
          k_mine, v_mine, ctx_c, out_send, rs_recv,
          cp_sems, ks_s, kr_s, vs_s, vr_s, rss_s, rsr_s, ags_s, agr_s):
    me = lax.axis_index("i")

    bar = pltpu.get_barrier_semaphore()
    for d in (1, 2, 3):
        pl.semaphore_signal(bar, inc=1,
                            device_id=(lax.rem(me + d, N_DEV),),
                            device_id_type=pl.DeviceIdType.MESH)
    pl.semaphore_wait(bar, 3)

    cp_k = pltpu.make_async_copy(kp_ref.at[pl.ds(me * HQ, HQ)],
                                 k_mine.at[0], cp_sems.at[0])
    cp_v = pltpu.make_async_copy(vp_ref.at[pl.ds(me * HQ, HQ)],
                                 v_mine.at[0], cp_sems.at[1])
    cp_k.start()
    cp_v.start()

    kv_rdmas = []
    for d in (1, 2, 3):
        p = lax.rem(me + d, N_DEV)
        rk = pltpu.make_async_remote_copy(
            src_ref=kp_ref.at[pl.ds(p * HQ, HQ)],
            dst_ref=k_mine.at[d],
            send_sem=ks_s.at[d - 1], recv_sem=kr_s.at[d - 1],
            device_id=(p,), device_id_type=pl.DeviceIdType.MESH)
        rk.start()
        rv = pltpu.make_async_remote_copy(
            src_ref=vp_ref.at[pl.ds(p * HQ, HQ)],
            dst_ref=v_mine.at[d],
            send_sem=vs_s.at[d - 1], recv_sem=vr_s.at[d - 1],
            device_id=(p,), device_id_type=pl.DeviceIdType.MESH)
        rv.start()
        kv_rdmas += [rk, rv]

    cp_k.wait()
    cp_v.wait()
    for r in kv_rdmas:
        r.wait_recv()

    for c in range(NCLS):
        xc = x_ref[:, c].reshape(CLS_ROWS, DM)
        qc = lax.dot_general(xc, wq_ref[...], (((1,), (0,)), ((), ())),
                             preferred_element_type=jnp.float32)
        qc = (qc * SCALE).astype(jnp.bfloat16)
        for h in range(HQ):
            qh = qc[:, h * DH:(h + 1) * DH]
            s_blocks = []
            for s in range(N_DEV):
                ks = k_mine[s, h, c]
                s_blocks.append(
                    lax.dot_general(qh, ks, (((1,), (0,)), ((), ())),
                                    preferred_element_type=jnp.float32))
            m = s_blocks[0].max(axis=1, keepdims=True)
            for t in s_blocks[1:]:
                m = jnp.maximum(m, t.max(axis=1, keepdims=True))
            acc = jnp.zeros((CLS_ROWS, DH), jnp.float32)
            den = jnp.zeros((CLS_ROWS, 1), jnp.float32)
            for s in range(N_DEV):
                e = jnp.exp(s_blocks[s] - m)
                den = den + e.sum(axis=1, keepdims=True)
                acc = acc + lax.dot_general(
                    e.astype(jnp.bfloat16), v_mine[s, h, c],
                    (((1,), (0,)), ((), ())),
                    preferred_element_type=jnp.float32)
            ctx_c[:, h * DH:(h + 1) * DH] = (acc / den).astype(jnp.bfloat16)
        outc = lax.dot_general(ctx_c[...], wo_ref[...],
                               (((1,), (0,)), ((), ())),
                               preferred_element_type=jnp.float32)
        out_send[:, c] = outc.astype(jnp.bfloat16).reshape(NG, BLK, DM)

    rs_list = []
    for d in (1, 2, 3):
        p = lax.rem(me + d, N_DEV)
        r = pltpu.make_async_remote_copy(
            src_ref=out_send.at[pl.ds(p * 2, 2)],
            dst_ref=rs_recv.at[d - 1],
            send_sem=rss_s.at[d - 1], recv_sem=rsr_s.at[d - 1],
            device_id=(p,), device_id_type=pl.DeviceIdType.MESH)
        r.start()
        rs_list.append(r)
    for r in rs_list:
        r.wait_recv()

    red = out_send[pl.ds(me * 2, 2)].reshape(QUARTER, DM).astype(jnp.float32)
    for i in range(3):
        red = red + rs_recv[i].reshape(QUARTER, DM).astype(jnp.float32)
    out_ref[pl.ds(me * QUARTER, QUARTER), :] = red.astype(jnp.bfloat16)

    ag_list = []
    for d in (1, 2, 3):
        p = lax.rem(me + d, N_DEV)
        r = pltpu.make_async_remote_copy(
            src_ref=out_ref.at[pl.ds(me * QUARTER, QUARTER)],
            dst_ref=out_ref.at[pl.ds(me * QUARTER, QUARTER)],
            send_sem=ags_s.at[d - 1], recv_sem=agr_s.at[d - 1],
            device_id=(p,), device_id_type=pl.DeviceIdType.MESH)
        r.start()
        ag_list.append(r)
    for r in ag_list:
        r.wait_recv()
    for r in kv_rdmas + rs_list + ag_list:
        r.wait_send()


def kernel(x, Wq, K_ext, V_ext, Wo):
    xb = x[0].astype(jnp.bfloat16).reshape(NG, NCLS, BLK, DM)
    wq = Wq.astype(jnp.bfloat16)
    wo = Wo.astype(jnp.bfloat16)
    kp = (K_ext[0].astype(jnp.bfloat16)
          .reshape(NG, NCLS, BLK, H_ALL, DH)
          .transpose(3, 1, 4, 0, 2)
          .reshape(H_ALL, NCLS, DH, CLS_ROWS))
    vp = (V_ext[0].astype(jnp.bfloat16)
          .reshape(NG, NCLS, BLK, H_ALL, DH)
          .transpose(3, 1, 0, 2, 4)
          .reshape(H_ALL, NCLS, CLS_ROWS, DH))

    out = pl.pallas_call(
        _body,
        out_shape=jax.ShapeDtypeStruct((SQ, DM), jnp.bfloat16),
        in_specs=[
            pl.BlockSpec(memory_space=pltpu.MemorySpace.VMEM),
            pl.BlockSpec(memory_space=pltpu.MemorySpace.VMEM),
            pl.BlockSpec(memory_space=pltpu.MemorySpace.HBM),
            pl.BlockSpec(memory_space=pltpu.MemorySpace.HBM),
            pl.BlockSpec(memory_space=pltpu.MemorySpace.VMEM),
        ],
        out_specs=pl.BlockSpec(memory_space=pltpu.MemorySpace.VMEM),
        scratch_shapes=[
            pltpu.VMEM((N_DEV, HQ, NCLS, DH, CLS_ROWS), jnp.bfloat16),
            pltpu.VMEM((N_DEV, HQ, NCLS, CLS_ROWS, DH), jnp.bfloat16),
            pltpu.VMEM((CLS_ROWS, DM), jnp.bfloat16),
            pltpu.VMEM((NG, NCLS, BLK, DM), jnp.bfloat16),
            pltpu.VMEM((3, 2, NCLS, BLK, DM), jnp.bfloat16),
            pltpu.SemaphoreType.DMA((2,)),
            pltpu.SemaphoreType.DMA((3,)),
            pltpu.SemaphoreType.DMA((3,)),
            pltpu.SemaphoreType.DMA((3,)),
            pltpu.SemaphoreType.DMA((3,)),
            pltpu.SemaphoreType.DMA((3,)),
            pltpu.SemaphoreType.DMA((3,)),
            pltpu.SemaphoreType.DMA((3,)),
            pltpu.SemaphoreType.DMA((3,)),
        ],
        compiler_params=pltpu.CompilerParams(collective_id=0),
    )(xb, wq, kp, vp, wo)
    return out.reshape(1, SQ, DM).astype(jnp.float32)


# baseline (device time: 447256 ns/iter reference)
import jax
import jax.numpy as jnp
from jax import lax
from jax.experimental import pallas as pl
from jax.experimental.pallas import tpu as pltpu

N_DEV = 4
HQ = 8
H_ALL = 32
DH = 128
SQ = 2048
NCLS = 4
NG = 8
BLK = 64
CLS_ROWS = NG * BLK
QUARTER = SQ // N_DEV
DM = 1024
SCALE = 0.08838834764831843


def _body(x_ref, wq_ref, kp_ref, vp_ref, wo_ref, out_ref,
          k_mine, v_mine, outc_ref, s_ref, rs_recv,
          cp_sems, ks_s, kr_s, vs_s, vr_s, rss_s, rsr_s, ags_s, agr_s):
    me = lax.axis_index("i")

    bar = pltpu.get_barrier_semaphore()
    for d in (1, 2, 3):
        pl.semaphore_signal(bar, inc=1,
                            device_id=(lax.rem(me + d, N_DEV),),
                            device_id_type=pl.DeviceIdType.MESH)
    pl.semaphore_wait(bar, 3)

    cp_k = pltpu.make_async_copy(kp_ref.at[pl.ds(me * HQ, HQ)],
                                 k_mine.at[0], cp_sems.at[0])
    cp_v = pltpu.make_async_copy(vp_ref.at[pl.ds(me * HQ, HQ)],
                                 v_mine.at[0], cp_sems.at[1])
    cp_k.start()
    cp_v.start()

    kv_rdmas = []
    for d in (1, 2, 3):
        p = lax.rem(me + d, N_DEV)
        rk = pltpu.make_async_remote_copy(
            src_ref=kp_ref.at[pl.ds(p * HQ, HQ)],
            dst_ref=k_mine.at[d],
            send_sem=ks_s.at[d - 1], recv_sem=kr_s.at[d - 1],
            device_id=(p,), device_id_type=pl.DeviceIdType.MESH)
        rk.start()
        rv = pltpu.make_async_remote_copy(
            src_ref=vp_ref.at[pl.ds(p * HQ, HQ)],
            dst_ref=v_mine.at[d],
            send_sem=vs_s.at[d - 1], recv_sem=vr_s.at[d - 1],
            device_id=(p,), device_id_type=pl.DeviceIdType.MESH)
        rv.start()
        kv_rdmas += [rk, rv]

    cp_k.wait()
    cp_v.wait()
    for r in kv_rdmas:
        r.wait_recv()

    def attn_step(i, carry):
        c = i // HQ
        h = lax.rem(i, HQ)
        xc = x_ref[:, c].reshape(CLS_ROWS, DM)
        qh = lax.dot_general(xc, wq_ref[h], (((1,), (0,)), ((), ())),
                             preferred_element_type=jnp.float32)
        qh = (qh * SCALE).astype(jnp.bfloat16)
        for s in range(N_DEV):
            ks = k_mine[s, h, c]
            s_ref[:, s * CLS_ROWS:(s + 1) * CLS_ROWS] = lax.dot_general(
                qh, ks, (((1,), (0,)), ((), ())),
                preferred_element_type=jnp.float32)
        m = s_ref[...].max(axis=1, keepdims=True)
        acc = jnp.zeros((CLS_ROWS, DH), jnp.float32)
        den = jnp.zeros((CLS_ROWS, 1), jnp.float32)
        for s in range(N_DEV):
            e = jnp.exp(s_ref[:, s * CLS_ROWS:(s + 1) * CLS_ROWS] - m)
            den = den + e.sum(axis=1, keepdims=True)
            acc = acc + lax.dot_general(
                e.astype(jnp.bfloat16), v_mine[s, h, c],
                (((1,), (0,)), ((), ())),
                preferred_element_type=jnp.float32)
        ctx_h = (acc / den).astype(jnp.bfloat16)
        contrib = lax.dot_general(ctx_h, wo_ref[h], (((1,), (0,)), ((), ())),
                                  preferred_element_type=jnp.float32)

        @pl.when(h == 0)
        def _():
            outc_ref[...] = contrib

        @pl.when(h != 0)
        def _():
            outc_ref[...] = outc_ref[...] + contrib

        @pl.when(h == HQ - 1)
        def _():
            out_ref[:, c] = outc_ref[...].astype(jnp.bfloat16).reshape(
                NG, BLK, DM)

        return carry

    lax.fori_loop(0, NCLS * HQ, attn_step, 0)

    rs_list = []
    for d in (1, 2, 3):
        p = lax.rem(me + d, N_DEV)
        r = pltpu.make_async_remote_copy(
            src_ref=out_ref.at[pl.ds(p * 2, 2)],
            dst_ref=rs_recv.at[d - 1],
            send_sem=rss_s.at[d - 1], recv_sem=rsr_s.at[d - 1],
            device_id=(p,), device_id_type=pl.DeviceIdType.MESH)
        r.start()
        rs_list.append(r)
    for r in rs_list:
        r.wait_recv()

    red = out_ref[pl.ds(me * 2, 2)].reshape(QUARTER, DM).astype(jnp.float32)
    for i in range(3):
        red = red + rs_recv[i].reshape(QUARTER, DM).astype(jnp.float32)
    out_ref[pl.ds(me * 2, 2)] = red.astype(jnp.bfloat16).reshape(
        2, NCLS, BLK, DM)

    ag_list = []
    for d in (1, 2, 3):
        p = lax.rem(me + d, N_DEV)
        r = pltpu.make_async_remote_copy(
            src_ref=out_ref.at[pl.ds(me * 2, 2)],
            dst_ref=out_ref.at[pl.ds(me * 2, 2)],
            send_sem=ags_s.at[d - 1], recv_sem=agr_s.at[d - 1],
            device_id=(p,), device_id_type=pl.DeviceIdType.MESH)
        r.start()
        ag_list.append(r)
    for r in ag_list:
        r.wait_recv()
    for r in kv_rdmas + rs_list + ag_list:
        r.wait_send()


def kernel(x, Wq, K_ext, V_ext, Wo):
    xb = x[0].astype(jnp.bfloat16).reshape(NG, NCLS, BLK, DM)
    wq = Wq.astype(jnp.bfloat16).reshape(DM, HQ, DH).transpose(1, 0, 2)
    wo = Wo.astype(jnp.bfloat16).reshape(HQ, DH, DM)
    kp = (K_ext[0].astype(jnp.bfloat16)
          .reshape(NG, NCLS, BLK, H_ALL, DH)
          .transpose(3, 1, 4, 0, 2)
          .reshape(H_ALL, NCLS, DH, CLS_ROWS))
    vp = (V_ext[0].astype(jnp.bfloat16)
          .reshape(NG, NCLS, BLK, H_ALL, DH)
          .transpose(3, 1, 0, 2, 4)
          .reshape(H_ALL, NCLS, CLS_ROWS, DH))

    out = pl.pallas_call(
        _body,
        out_shape=jax.ShapeDtypeStruct((NG, NCLS, BLK, DM), jnp.bfloat16),
        in_specs=[
            pl.BlockSpec(memory_space=pltpu.MemorySpace.VMEM),
            pl.BlockSpec(memory_space=pltpu.MemorySpace.VMEM),
            pl.BlockSpec(memory_space=pltpu.MemorySpace.HBM),
            pl.BlockSpec(memory_space=pltpu.MemorySpace.HBM),
            pl.BlockSpec(memory_space=pltpu.MemorySpace.VMEM),
        ],
        out_specs=pl.BlockSpec(memory_space=pltpu.MemorySpace.VMEM),
        scratch_shapes=[
            pltpu.VMEM((N_DEV, HQ, NCLS, DH, CLS_ROWS), jnp.bfloat16),
            pltpu.VMEM((N_DEV, HQ, NCLS, CLS_ROWS, DH), jnp.bfloat16),
            pltpu.VMEM((CLS_ROWS, DM), jnp.float32),
            pltpu.VMEM((CLS_ROWS, N_DEV * CLS_ROWS), jnp.float32),
            pltpu.VMEM((3, 2, NCLS, BLK, DM), jnp.bfloat16),
            pltpu.SemaphoreType.DMA((2,)),
            pltpu.SemaphoreType.DMA((3,)),
            pltpu.SemaphoreType.DMA((3,)),
            pltpu.SemaphoreType.DMA((3,)),
            pltpu.SemaphoreType.DMA((3,)),
            pltpu.SemaphoreType.DMA((3,)),
            pltpu.SemaphoreType.DMA((3,)),
            pltpu.SemaphoreType.DMA((3,)),
            pltpu.SemaphoreType.DMA((3,)),
        ],
        compiler_params=pltpu.CompilerParams(
            collective_id=0, vmem_limit_bytes=100 * 1024 * 1024),
    )(xb, wq, kp, vp, wo)
    return out.reshape(1, SQ, DM).astype(jnp.float32)


# device time: 395544 ns/iter; 1.1307x vs baseline; 1.1307x over previous
import jax
import jax.numpy as jnp
from jax import lax
from jax.experimental import pallas as pl
from jax.experimental.pallas import tpu as pltpu

N_DEV = 4
HQ = 8
H_ALL = 32
DH = 128
SQ = 2048
NCLS = 4
NG = 8
BLK = 64
CLS_ROWS = NG * BLK
QUARTER = SQ // N_DEV
DM = 1024
SCALE = 0.08838834764831843
SLOT_ORDER = (0, 1, 3, 2)


def _body(x_ref, wq_ref, kp_ref, vp_ref, wo_ref,
          out_ref, k_mine, v_mine,
          q_ref, kst, vst, acc_ref, den_ref, outc_ref,
          rs_recv,
          ks_s, kr_s, vs_s, vr_s, stk_s, stv_s, rss_s, rsr_s, ags_s, agr_s):
    me = lax.axis_index("i")

    bar = pltpu.get_barrier_semaphore()
    for d in (1, 2, 3):
        pl.semaphore_signal(bar, inc=1,
                            device_id=(lax.rem(me + d, N_DEV),),
                            device_id_type=pl.DeviceIdType.MESH)
    pl.semaphore_wait(bar, 3)

    recv_desc = {}
    kv_rdmas = []
    for d in (1, 2, 3):
        p = lax.rem(me + d, N_DEV)
        rk = pltpu.make_async_remote_copy(
            src_ref=kp_ref.at[:, pl.ds(p * HQ, HQ)],
            dst_ref=k_mine.at[d],
            send_sem=ks_s.at[d - 1], recv_sem=kr_s.at[d - 1],
            device_id=(p,), device_id_type=pl.DeviceIdType.MESH)
        rk.start()
        rv = pltpu.make_async_remote_copy(
            src_ref=vp_ref.at[:, pl.ds(p * HQ, HQ)],
            dst_ref=v_mine.at[d],
            send_sem=vs_s.at[d - 1], recv_sem=vr_s.at[d - 1],
            device_id=(p,), device_id_type=pl.DeviceIdType.MESH)
        rv.start()
        kv_rdmas += [rk, rv]
        recv_desc[d] = (rk, rv)

    def stage(s, c, b):
        if s == 0:
            src_k = kp_ref.at[c, pl.ds(me * HQ, HQ)]
            src_v = vp_ref.at[c, pl.ds(me * HQ, HQ)]
        else:
            src_k = k_mine.at[s, c]
            src_v = v_mine.at[s, c]
        ck = pltpu.make_async_copy(src_k, kst.at[b], stk_s.at[b])
        cv = pltpu.make_async_copy(src_v, vst.at[b], stv_s.at[b])
        ck.start()
        cv.start()
        return ck, cv

    pending = stage(0, 0, 0)

    def qstep(i, carry):
        c = i // HQ
        h = lax.rem(i, HQ)
        xc = x_ref[:, c].reshape(CLS_ROWS, DM)
        qh = lax.dot_general(xc, wq_ref[h], (((1,), (0,)), ((), ())),
                             preferred_element_type=jnp.float32)
        q_ref[c, h] = (qh * SCALE).astype(jnp.bfloat16)
        return carry

    lax.fori_loop(0, NCLS * HQ, qstep, 0)

    pieces = [(si, s, c) for si, s in enumerate(SLOT_ORDER)
              for c in range(NCLS)]
    for t, (si, s, c) in enumerate(pieces):
        ck, cv = pending
        ck.wait()
        cv.wait()
        if t + 1 < len(pieces):
            si2, s2, c2 = pieces[t + 1]
            if c2 == 0 and s2 != 0:
                rk, rv = recv_desc[s2]
                rk.wait_recv()
                rv.wait_recv()
            pending = stage(s2, c2, (t + 1) % 2)

        def astep(h, carry, b=t % 2, c=c, first=(si == 0)):
            e = jnp.exp(lax.dot_general(
                q_ref[c, h], kst[b, h], (((1,), (0,)), ((), ())),
                preferred_element_type=jnp.float32))
            dsum = jnp.sum(e, axis=1, keepdims=True)
            pv = lax.dot_general(e.astype(jnp.bfloat16), vst[b, h],
                                 (((1,), (0,)), ((), ())),
                                 preferred_element_type=jnp.float32)
            if first:
                den_ref[c, h] = dsum
                acc_ref[c, h] = pv
            else:
                den_ref[c, h] = den_ref[c, h] + dsum
                acc_ref[c, h] = acc_ref[c, h] + pv
            return carry

        lax.fori_loop(0, HQ, astep, 0)

    def fstep(i, carry):
        c = i // HQ
        h = lax.rem(i, HQ)
        ctx = (acc_ref[c, h] / den_ref[c, h]).astype(jnp.bfloat16)
        contrib = lax.dot_general(ctx, wo_ref[h], (((1,), (0,)), ((), ())),
                                  preferred_element_type=jnp.float32)

        @pl.when(h == 0)
        def _():
            outc_ref[...] = contrib

        @pl.when(h != 0)
        def _():
            outc_ref[...] = outc_ref[...] + contrib

        @pl.when(h == HQ - 1)
        def _():
            out_ref[:, c] = outc_ref[...].astype(jnp.bfloat16).reshape(
                NG, BLK, DM)

        return carry

    lax.fori_loop(0, NCLS * HQ, fstep, 0)

    rs_list = []
    for d in (1, 2, 3):
        p = lax.rem(me + d, N_DEV)
        r = pltpu.make_async_remote_copy(
            src_ref=out_ref.at[pl.ds(p * 2, 2)],
            dst_ref=rs_recv.at[d - 1],
            send_sem=rss_s.at[d - 1], recv_sem=rsr_s.at[d - 1],
            device_id=(p,), device_id_type=pl.DeviceIdType.MESH)
        r.start()
        rs_list.append(r)
    for r in rs_list:
        r.wait_recv()

    red = out_ref[pl.ds(me * 2, 2)].reshape(QUARTER, DM).astype(jnp.float32)
    for i in range(3):
        red = red + rs_recv[i].reshape(QUARTER, DM).astype(jnp.float32)
    out_ref[pl.ds(me * 2, 2)] = red.astype(jnp.bfloat16).reshape(
        2, NCLS, BLK, DM)

    ag_list = []
    for d in (1, 2, 3):
        p = lax.rem(me + d, N_DEV)
        r = pltpu.make_async_remote_copy(
            src_ref=out_ref.at[pl.ds(me * 2, 2)],
            dst_ref=out_ref.at[pl.ds(me * 2, 2)],
            send_sem=ags_s.at[d - 1], recv_sem=agr_s.at[d - 1],
            device_id=(p,), device_id_type=pl.DeviceIdType.MESH)
        r.start()
        ag_list.append(r)
    for r in ag_list:
        r.wait_recv()
    for r in kv_rdmas + rs_list + ag_list:
        r.wait_send()


def kernel(x, Wq, K_ext, V_ext, Wo):
    xb = x[0].astype(jnp.bfloat16).reshape(NG, NCLS, BLK, DM)
    wq = Wq.astype(jnp.bfloat16).reshape(DM, HQ, DH).transpose(1, 0, 2)
    wo = Wo.astype(jnp.bfloat16).reshape(HQ, DH, DM)
    kp = (K_ext[0].astype(jnp.bfloat16)
          .reshape(NG, NCLS, BLK, H_ALL, DH)
          .transpose(1, 3, 4, 0, 2)
          .reshape(NCLS, H_ALL, DH, CLS_ROWS))
    vp = (V_ext[0].astype(jnp.bfloat16)
          .reshape(NG, NCLS, BLK, H_ALL, DH)
          .transpose(1, 3, 0, 2, 4)
          .reshape(NCLS, H_ALL, CLS_ROWS, DH))

    out, _, _ = pl.pallas_call(
        _body,
        out_shape=(
            jax.ShapeDtypeStruct((NG, NCLS, BLK, DM), jnp.bfloat16),
            jax.ShapeDtypeStruct((N_DEV, NCLS, HQ, DH, CLS_ROWS),
                                 jnp.bfloat16),
            jax.ShapeDtypeStruct((N_DEV, NCLS, HQ, CLS_ROWS, DH),
                                 jnp.bfloat16),
        ),
        in_specs=[
            pl.BlockSpec(memory_space=pltpu.MemorySpace.VMEM),
            pl.BlockSpec(memory_space=pltpu.MemorySpace.VMEM),
            pl.BlockSpec(memory_space=pltpu.MemorySpace.HBM),
            pl.BlockSpec(memory_space=pltpu.MemorySpace.HBM),
            pl.BlockSpec(memory_space=pltpu.MemorySpace.VMEM),
        ],
        out_specs=(
            pl.BlockSpec(memory_space=pltpu.MemorySpace.VMEM),
            pl.BlockSpec(memory_space=pltpu.MemorySpace.HBM),
            pl.BlockSpec(memory_space=pltpu.MemorySpace.HBM),
        ),
        scratch_shapes=[
            pltpu.VMEM((NCLS, HQ, CLS_ROWS, DH), jnp.bfloat16),
            pltpu.VMEM((2, HQ, DH, CLS_ROWS), jnp.bfloat16),
            pltpu.VMEM((2, HQ, CLS_ROWS, DH), jnp.bfloat16),
            pltpu.VMEM((NCLS, HQ, CLS_ROWS, DH), jnp.float32),
            pltpu.VMEM((NCLS, HQ, CLS_ROWS, 1), jnp.float32),
            pltpu.VMEM((CLS_ROWS, DM), jnp.float32),
            pltpu.VMEM((3, 2, NCLS, BLK, DM), jnp.bfloat16),
            pltpu.SemaphoreType.DMA((3,)),
            pltpu.SemaphoreType.DMA((3,)),
            pltpu.SemaphoreType.DMA((3,)),
            pltpu.SemaphoreType.DMA((3,)),
            pltpu.SemaphoreType.DMA((2,)),
            pltpu.SemaphoreType.DMA((2,)),
            pltpu.SemaphoreType.DMA((3,)),
            pltpu.SemaphoreType.DMA((3,)),
            pltpu.SemaphoreType.DMA((3,)),
            pltpu.SemaphoreType.DMA((3,)),
        ],
        compiler_params=pltpu.CompilerParams(
            collective_id=0, vmem_limit_bytes=100 * 1024 * 1024),
    )(xb, wq, kp, vp, wo)
    return out.reshape(1, SQ, DM).astype(jnp.float32)
